# baseline (speedup 1.0000x reference)
import functools
import math

import jax
import jax.numpy as jnp
from jax import lax
from jax.experimental import pallas as pl
from jax.experimental.pallas import tpu as pltpu
from jax.experimental.pallas import tpu_sc as plsc

D_MODEL = 128
SCALE = math.sqrt(float(D_MODEL))

NUM_CORES = 2
NUM_SUBCORES = 16
NUM_LANES = 16
NW = NUM_CORES * NUM_SUBCORES

CHUNK = 128


def _build_gather(n_chunks: int):
    assert n_chunks % 2 == 0
    mesh = plsc.VectorSubcoreMesh(core_axis_name="c", subcore_axis_name="s")
    total_rows = NW * n_chunks * CHUNK

    @functools.partial(
        pl.kernel,
        mesh=mesh,
        out_type=jax.ShapeDtypeStruct((total_rows, D_MODEL), jnp.float32),
        scratch_types=[
            pltpu.VMEM((n_chunks, CHUNK), jnp.int32),
            pltpu.VMEM((CHUNK, D_MODEL), jnp.float32),
            pltpu.VMEM((CHUNK, D_MODEL), jnp.float32),
            pltpu.VMEM((CHUNK, D_MODEL), jnp.float32),
            pltpu.VMEM((CHUNK, D_MODEL), jnp.float32),
            pltpu.SemaphoreType.DMA,
            pltpu.SemaphoreType.DMA,
            pltpu.SemaphoreType.DMA,
            pltpu.SemaphoreType.DMA,
        ],
    )
    def k(table_hbm, idx_hbm, out_hbm, idx_v,
          ibuf0, ibuf1, obuf0, obuf1, isem0, isem1, osem0, osem1):
        wid = lax.axis_index("s") * NUM_CORES + lax.axis_index("c")
        base = wid * (n_chunks * CHUNK)
        pltpu.sync_copy(idx_hbm.at[wid], idx_v)

        def start_gather(j, ibuf, isem):
            pltpu.make_async_copy(table_hbm.at[idx_v.at[j]], ibuf, isem).start()

        def wait_gather(j, ibuf, isem):
            pltpu.make_async_copy(table_hbm.at[idx_v.at[j]], ibuf, isem).wait()

        def start_store(j, obuf, osem):
            dst = out_hbm.at[pl.ds(base + j * CHUNK, CHUNK)]
            pltpu.make_async_copy(obuf, dst, osem).start()

        def wait_store(obuf, osem):
            dst = out_hbm.at[pl.ds(base, CHUNK)]
            pltpu.make_async_copy(obuf, dst, osem).wait()

        def scale(ibuf, obuf):
            @plsc.parallel_loop(0, CHUNK, unroll=4)
            def _(r):
                for c in range(D_MODEL // NUM_LANES):
                    sl = pl.ds(c * NUM_LANES, NUM_LANES)
                    obuf.at[r, sl][...] = ibuf.at[r, sl][...] * SCALE

        start_gather(0, ibuf0, isem0)
        start_gather(1, ibuf1, isem1)

        @pl.loop(0, n_chunks, step=2)
        def _(j):
            def half(jj, ibuf, isem, obuf, osem):
                wait_gather(jj, ibuf, isem)

                @pl.when(jj >= 2)
                def _():
                    wait_store(obuf, osem)

                scale(ibuf, obuf)

                @pl.when(jj + 2 < n_chunks)
                def _():
                    start_gather(jj + 2, ibuf, isem)

                start_store(jj, obuf, osem)

            half(j, ibuf0, isem0, obuf0, osem0)
            half(j + 1, ibuf1, isem1, obuf1, osem1)

        wait_store(obuf0, osem0)
        wait_store(obuf1, osem1)

    return k


def kernel(x, table):
    b, l = x.shape
    vocab, d = table.shape
    assert d == D_MODEL
    n_total = b * l
    assert n_total % (NW * CHUNK) == 0
    n_chunks = n_total // (NW * CHUNK)
    idx = x.reshape(NW, n_chunks, CHUNK).astype(jnp.int32)
    out = _build_gather(n_chunks)(table, idx)
    return out.reshape(b, l, d)


# speedup vs baseline: 1.0127x; 1.0127x over previous
import functools
import math

import jax
import jax.numpy as jnp
from jax import lax
from jax.experimental import pallas as pl
from jax.experimental.pallas import tpu as pltpu
from jax.experimental.pallas import tpu_sc as plsc

D_MODEL = 128
SCALE = math.sqrt(float(D_MODEL))

NUM_CORES = 2
NUM_SUBCORES = 16
NUM_LANES = 16
NW = NUM_CORES * NUM_SUBCORES

CHUNK = 128


def _build_gather(n_chunks: int):
    assert n_chunks % 2 == 0
    mesh = plsc.VectorSubcoreMesh(core_axis_name="c", subcore_axis_name="s")
    total_rows = NW * n_chunks * CHUNK

    DEPTH = 3
    n_main = (n_chunks // DEPTH) * DEPTH

    @functools.partial(
        pl.kernel,
        mesh=mesh,
        out_type=jax.ShapeDtypeStruct((total_rows, D_MODEL), jnp.float32),
        scratch_types=[
            pltpu.VMEM((n_chunks, CHUNK), jnp.int32),
        ]
        + [pltpu.VMEM((CHUNK, D_MODEL), jnp.float32)] * (2 * DEPTH)
        + [pltpu.SemaphoreType.DMA] * (2 * DEPTH),
    )
    def k(table_hbm, idx_hbm, out_hbm, idx_v, *bufs_sems):
        ibufs = bufs_sems[0:DEPTH]
        obufs = bufs_sems[DEPTH:2 * DEPTH]
        isems = bufs_sems[2 * DEPTH:3 * DEPTH]
        osems = bufs_sems[3 * DEPTH:4 * DEPTH]

        wid = lax.axis_index("s") * NUM_CORES + lax.axis_index("c")
        base = wid * (n_chunks * CHUNK)
        pltpu.sync_copy(idx_hbm.at[wid], idx_v)

        def start_gather(j, s):
            pltpu.make_async_copy(
                table_hbm.at[idx_v.at[j]], ibufs[s], isems[s]).start()

        def wait_gather(j, s):
            pltpu.make_async_copy(
                table_hbm.at[idx_v.at[j]], ibufs[s], isems[s]).wait()

        def start_store(j, s):
            dst = out_hbm.at[pl.ds(base + j * CHUNK, CHUNK)]
            pltpu.make_async_copy(obufs[s], dst, osems[s]).start()

        def wait_store(s):
            dst = out_hbm.at[pl.ds(base, CHUNK)]
            pltpu.make_async_copy(obufs[s], dst, osems[s]).wait()

        def scale(s):
            ibuf, obuf = ibufs[s], obufs[s]

            @plsc.parallel_loop(0, CHUNK, unroll=4)
            def _(r):
                for c in range(D_MODEL // NUM_LANES):
                    sl = pl.ds(c * NUM_LANES, NUM_LANES)
                    obuf.at[r, sl][...] = ibuf.at[r, sl][...] * SCALE

        for s in range(DEPTH):
            start_gather(s, s)

        @pl.loop(0, n_main, step=DEPTH)
        def _(j):
            for s in range(DEPTH):
                jj = j + s
                wait_gather(jj, s)

                @pl.when(jj >= DEPTH)
                def _():
                    wait_store(s)

                scale(s)
                if n_main + s >= n_chunks:
                    @pl.when(jj + DEPTH < n_chunks)
                    def _():
                        start_gather(jj + DEPTH, s)
                else:
                    start_gather(jj + DEPTH, s)
                start_store(jj, s)

        for j in range(n_main, n_chunks):
            s = j % DEPTH
            wait_gather(j, s)
            wait_store(s)
            scale(s)
            start_store(j, s)

        for s in range(DEPTH):
            wait_store(s)

    return k


def kernel(x, table):
    b, l = x.shape
    vocab, d = table.shape
    assert d == D_MODEL
    n_total = b * l
    assert n_total % (NW * CHUNK) == 0
    n_chunks = n_total // (NW * CHUNK)
    idx = x.reshape(NW, n_chunks, CHUNK).astype(jnp.int32)
    out = _build_gather(n_chunks)(table, idx)
    return out.reshape(b, l, d)


# speedup vs baseline: 1.0142x; 1.0015x over previous
import functools
import math

import jax
import jax.numpy as jnp
from jax import lax
from jax.experimental import pallas as pl
from jax.experimental.pallas import tpu as pltpu
from jax.experimental.pallas import tpu_sc as plsc

D_MODEL = 128
SCALE = math.sqrt(float(D_MODEL))

NUM_CORES = 2
NUM_SUBCORES = 16
NUM_LANES = 16
NW = NUM_CORES * NUM_SUBCORES

CHUNK = 128


def _build_gather(n_chunks: int):
    assert n_chunks % 2 == 0
    mesh = plsc.VectorSubcoreMesh(core_axis_name="c", subcore_axis_name="s")
    total_rows = NW * n_chunks * CHUNK

    DEPTH = 3
    n_main = (n_chunks // DEPTH) * DEPTH

    @functools.partial(
        pl.kernel,
        mesh=mesh,
        out_type=jax.ShapeDtypeStruct((total_rows, D_MODEL), jnp.float32),
        scratch_types=[
            pltpu.VMEM((n_chunks * CHUNK,), jnp.int32),
        ]
        + [pltpu.VMEM((CHUNK, D_MODEL), jnp.float32)] * (2 * DEPTH)
        + [pltpu.SemaphoreType.DMA] * (2 * DEPTH),
    )
    def k(table_hbm, idx_hbm, out_hbm, idx_v, *bufs_sems):
        ibufs = bufs_sems[0:DEPTH]
        obufs = bufs_sems[DEPTH:2 * DEPTH]
        isems = bufs_sems[2 * DEPTH:3 * DEPTH]
        osems = bufs_sems[3 * DEPTH:4 * DEPTH]

        wid = lax.axis_index("s") * NUM_CORES + lax.axis_index("c")
        base = wid * (n_chunks * CHUNK)
        pltpu.sync_copy(idx_hbm.at[pl.ds(base, n_chunks * CHUNK)], idx_v)

        def start_gather(j, s):
            idx = idx_v.at[pl.ds(j * CHUNK, CHUNK)]
            pltpu.make_async_copy(table_hbm.at[idx], ibufs[s], isems[s]).start()

        def wait_gather(j, s):
            idx = idx_v.at[pl.ds(j * CHUNK, CHUNK)]
            pltpu.make_async_copy(table_hbm.at[idx], ibufs[s], isems[s]).wait()

        def start_store(j, s):
            dst = out_hbm.at[pl.ds(base + j * CHUNK, CHUNK)]
            pltpu.make_async_copy(obufs[s], dst, osems[s]).start()

        def wait_store(s):
            dst = out_hbm.at[pl.ds(base, CHUNK)]
            pltpu.make_async_copy(obufs[s], dst, osems[s]).wait()

        def scale(s):
            ibuf, obuf = ibufs[s], obufs[s]

            @plsc.parallel_loop(0, CHUNK, unroll=4)
            def _(r):
                for c in range(D_MODEL // NUM_LANES):
                    sl = pl.ds(c * NUM_LANES, NUM_LANES)
                    obuf.at[r, sl][...] = ibuf.at[r, sl][...] * SCALE

        for s in range(DEPTH):
            start_gather(s, s)

        @pl.loop(0, n_main, step=DEPTH)
        def _(j):
            for s in range(DEPTH):
                jj = j + s
                wait_gather(jj, s)

                @pl.when(jj >= DEPTH)
                def _():
                    wait_store(s)

                scale(s)
                if n_main + s >= n_chunks:
                    @pl.when(jj + DEPTH < n_chunks)
                    def _():
                        start_gather(jj + DEPTH, s)
                else:
                    start_gather(jj + DEPTH, s)
                start_store(jj, s)

        for j in range(n_main, n_chunks):
            s = j % DEPTH
            wait_gather(j, s)
            wait_store(s)
            scale(s)
            start_store(j, s)

        for s in range(DEPTH):
            wait_store(s)

    return k


def kernel(x, table):
    b, l = x.shape
    vocab, d = table.shape
    assert d == D_MODEL
    n_total = b * l
    assert n_total % (NW * CHUNK) == 0
    n_chunks = n_total // (NW * CHUNK)
    idx = x.reshape(n_total).astype(jnp.int32)
    out = _build_gather(n_chunks)(table, idx)
    return out.reshape(b, l, d)
